# baseline (device time: 13370 ns/iter reference)
import functools

import jax
import jax.numpy as jnp
from jax import lax
from jax.experimental import pallas as pl
from jax.experimental.pallas import tpu as pltpu

N_DEV = 4
M_PER = 256


def kernel(x, w_mat):
    k_total, k_per = x.shape
    _, n = w_mat.shape

    def body(x_ref, w_ref, out_ref, xs_ref, comm_ref, w_vmem,
             send_sems, recv_sems, w_sems):
        my = lax.axis_index("i")

        barrier_sem = pltpu.get_barrier_semaphore()
        for d in range(1, N_DEV):
            pl.semaphore_signal(
                barrier_sem, inc=1,
                device_id=((my + d) % N_DEV,),
                device_id_type=pl.DeviceIdType.MESH,
            )

        w_dmas = {}
        for slot, e in ((0, 0), (1, 1), (3, 3), (2, 2)):
            j = (my + e) % N_DEV
            dma = pltpu.make_async_copy(
                w_ref.at[pl.ds(j * k_per, k_per), :],
                w_vmem.at[slot],
                w_sems.at[slot],
            )
            dma.start()
            w_dmas[slot] = dma

        xs_ref[...] = x_ref[...].astype(jnp.bfloat16)

        pl.semaphore_wait(barrier_sem, N_DEV - 1)

        rdmas = {}
        for d in range(1, N_DEV):
            t = (my + d) % N_DEV
            slot = 3 - d
            rdma = pltpu.make_async_remote_copy(
                src_ref=xs_ref.at[pl.ds(t * M_PER, M_PER), :],
                dst_ref=comm_ref.at[slot],
                send_sem=send_sems.at[slot],
                recv_sem=recv_sems.at[slot],
                device_id=(t,),
                device_id_type=pl.DeviceIdType.MESH,
            )
            rdma.start()
            rdmas[slot] = rdma

        w_dmas[0].wait()
        acc = jnp.dot(
            x_ref[pl.ds(my * M_PER, M_PER), :],
            w_vmem[0],
            preferred_element_type=jnp.float32,
        )

        for e in (1, 3, 2):
            slot = e - 1
            rdmas[slot].wait_recv()
            w_dmas[e].wait()
            acc += jnp.dot(
                comm_ref[slot].astype(jnp.float32),
                w_vmem[e],
                preferred_element_type=jnp.float32,
            )

        out_ref[:, :] = acc * jax.nn.sigmoid(acc)

        for slot in range(N_DEV - 1):
            rdmas[slot].wait_send()

    return pl.pallas_call(
        body,
        out_shape=jax.ShapeDtypeStruct((M_PER, n), jnp.float32),
        in_specs=[
            pl.BlockSpec(memory_space=pltpu.VMEM),
            pl.BlockSpec(memory_space=pl.ANY),
        ],
        out_specs=pl.BlockSpec(memory_space=pltpu.VMEM),
        scratch_shapes=[
            pltpu.VMEM((k_total, k_per), jnp.bfloat16),
            pltpu.VMEM((N_DEV - 1, M_PER, k_per), jnp.bfloat16),
            pltpu.VMEM((N_DEV, k_per, n), jnp.float32),
            pltpu.SemaphoreType.DMA((N_DEV - 1,)),
            pltpu.SemaphoreType.DMA((N_DEV - 1,)),
            pltpu.SemaphoreType.DMA((N_DEV,)),
        ],
        compiler_params=pltpu.CompilerParams(collective_id=0),
    )(x, w_mat)


# device time: 12506 ns/iter; 1.0691x vs baseline; 1.0691x over previous
import jax
import jax.numpy as jnp
from jax import lax
from jax.experimental import pallas as pl
from jax.experimental.pallas import tpu as pltpu

N_DEV = 4
M_PER = 256


def kernel(x, w_mat):
    k_total, k_per = x.shape
    _, n = w_mat.shape

    def body(x_ref, w_ref, out_ref, xs_ref, comm_ref, send_sems, recv_sems):
        my = lax.axis_index("i")
        barrier_sem = pltpu.get_barrier_semaphore()

        rdmas = {}
        for d in (2, 1, 3):
            t = (my + d) % N_DEV
            slot = 3 - d
            xs_ref[slot] = x_ref[pl.ds(t * M_PER, M_PER), :].astype(
                jnp.bfloat16)
            rdma = pltpu.make_async_remote_copy(
                src_ref=xs_ref.at[slot],
                dst_ref=comm_ref.at[slot],
                send_sem=send_sems.at[slot],
                recv_sem=recv_sems.at[slot],
                device_id=(t,),
                device_id_type=pl.DeviceIdType.MESH,
            )
            rdma.start()
            rdmas[slot] = rdma

        acc = jnp.dot(
            x_ref[pl.ds(my * M_PER, M_PER), :],
            w_ref[pl.ds(my * k_per, k_per), :],
            preferred_element_type=jnp.float32,
        )

        for e in (1, 3, 2):
            slot = e - 1
            rdmas[slot].wait_recv()
            j = (my + e) % N_DEV
            pl.semaphore_signal(
                barrier_sem, inc=1,
                device_id=(j,),
                device_id_type=pl.DeviceIdType.MESH,
            )
            acc += jnp.dot(
                comm_ref[slot].astype(jnp.float32),
                w_ref[pl.ds(j * k_per, k_per), :],
                preferred_element_type=jnp.float32,
            )

        out_ref[:, :] = acc * jax.nn.sigmoid(acc)

        for slot in range(N_DEV - 1):
            rdmas[slot].wait_send()
        pl.semaphore_wait(barrier_sem, N_DEV - 1)

    return pl.pallas_call(
        body,
        out_shape=jax.ShapeDtypeStruct((M_PER, n), jnp.float32),
        in_specs=[
            pl.BlockSpec(memory_space=pltpu.VMEM),
            pl.BlockSpec(memory_space=pltpu.VMEM),
        ],
        out_specs=pl.BlockSpec(memory_space=pltpu.VMEM),
        scratch_shapes=[
            pltpu.VMEM((N_DEV - 1, M_PER, k_per), jnp.bfloat16),
            pltpu.VMEM((N_DEV - 1, M_PER, k_per), jnp.bfloat16),
            pltpu.SemaphoreType.DMA((N_DEV - 1,)),
            pltpu.SemaphoreType.DMA((N_DEV - 1,)),
        ],
        compiler_params=pltpu.CompilerParams(collective_id=0),
    )(x, w_mat)
